# baseline (speedup 1.0000x reference)
import jax
import jax.numpy as jnp
from jax.experimental import pallas as pl
from jax.experimental.pallas import tpu as pltpu

H = 32
W = 32
B = 8
G = 8
D = 64
HB = 8


def _logits_kernel(qh_ref, qw_ref, rh_ref, rw_ref, lh_ref, lw_ref):
    i = pl.program_id(0)
    posh = rh_ref[pl.ds(31 - i, H), :]
    poswt = jnp.transpose(rw_ref[pl.ds(31 - i, W), :])
    zero = jnp.zeros((H, D // 2), jnp.float32)
    rhs_h = jnp.concatenate([posh, zero], axis=1)
    rhs_w = jnp.concatenate([jnp.zeros((D // 2, W), jnp.float32), poswt],
                            axis=0)
    rhs_w4 = jnp.tile(rhs_w, (1, 4))
    xh = qh_ref[...].reshape(B * W * G, D)
    xw = qw_ref[...].reshape(B * H * G, D)
    lh = jax.lax.dot_general(xh, rhs_h, (((1,), (1,)), ((), ())),
                             preferred_element_type=jnp.float32)
    lw4 = jax.lax.dot_general(xw, rhs_w4, (((1,), (0,)), ((), ())),
                              preferred_element_type=jnp.float32)
    pad = jnp.zeros((B * W * G, 128 - H), jnp.float32)
    lh_ref[...] = jnp.concatenate([lh, pad], axis=1).astype(
        jnp.bfloat16).reshape(B, 1, W * G, 128)
    lw_ref[...] = lw4.reshape(B, H, 1, G, 128)


def _add_kernel(attn_ref, lh_ref, lw_ref, rep_ref, out_ref):
    rows = HB * W * G
    lh = lh_ref[...].reshape(rows, 128)
    lw4 = lw_ref[...].reshape(rows, 128)
    addend = jax.lax.dot_general(lh, rep_ref[...], (((1,), (0,)), ((), ())),
                                 preferred_element_type=jnp.float32)
    addend += pltpu.repeat(lw4, 8, axis=1)
    out_ref[...] = (attn_ref[...].reshape(rows, H * W) + addend).reshape(
        1, HB * W, G, H * W)


@jax.jit
def kernel(q, attn, rel_emb_h, rel_emb_w):
    QL = H * W
    q5 = q.reshape(B, H, W, G, D)
    rh = jnp.zeros((2 * H, H), jnp.float32).at[: 2 * H - 1].set(rel_emb_h)
    rw = jnp.zeros((2 * W, W), jnp.float32).at[: 2 * W - 1].set(rel_emb_w)

    lh_arr, lw_arr = pl.pallas_call(
        _logits_kernel,
        grid=(H,),
        in_specs=[
            pl.BlockSpec((B, 1, W, G, D), lambda i: (0, i, 0, 0, 0)),
            pl.BlockSpec((B, H, 1, G, D), lambda i: (0, 0, i, 0, 0)),
            pl.BlockSpec((2 * H, H), lambda i: (0, 0)),
            pl.BlockSpec((2 * W, W), lambda i: (0, 0)),
        ],
        out_specs=[
            pl.BlockSpec((B, 1, W * G, 128), lambda i: (0, i, 0, 0)),
            pl.BlockSpec((B, H, 1, G, 128), lambda i: (0, 0, i, 0, 0)),
        ],
        out_shape=[
            jax.ShapeDtypeStruct((B, H, W * G, 128), jnp.bfloat16),
            jax.ShapeDtypeStruct((B, H, W, G, 128), jnp.float32),
        ],
        compiler_params=pltpu.CompilerParams(
            dimension_semantics=("parallel",)),
        name="relpos_logits",
    )(q5, q5, rh, rw)

    j = jnp.arange(QL)
    rep = jnp.zeros((128, QL), jnp.float32).at[:H].set(
        (j[None, :] // W == jnp.arange(H)[:, None]).astype(jnp.float32)
    ).astype(jnp.bfloat16)

    out = pl.pallas_call(
        _add_kernel,
        grid=(B, H // HB),
        in_specs=[
            pl.BlockSpec((1, HB * W, G, QL), lambda b, h: (b, h, 0, 0)),
            pl.BlockSpec((1, HB, W * G, 128), lambda b, h: (b, h, 0, 0)),
            pl.BlockSpec((1, HB, W, G, 128), lambda b, h: (b, h, 0, 0, 0)),
            pl.BlockSpec((128, QL), lambda b, h: (0, 0)),
        ],
        out_specs=pl.BlockSpec((1, HB * W, G, QL), lambda b, h: (b, h, 0, 0)),
        out_shape=jax.ShapeDtypeStruct((B, QL, G, QL), jnp.float32),
        compiler_params=pltpu.CompilerParams(
            dimension_semantics=("parallel", "arbitrary"),
            vmem_limit_bytes=52 * 1024 * 1024),
        name="relpos_add",
    )(attn, lh_arr, lw_arr, rep)
    return out


# speedup vs baseline: 1.0443x; 1.0443x over previous
import jax
import jax.numpy as jnp
from jax.experimental import pallas as pl
from jax.experimental.pallas import tpu as pltpu

H = 32
W = 32
B = 8
G = 8
D = 64
OG = 4
HB = 8


def _logits_kernel(qh_ref, qw_ref, rh_ref, rw_ref, lh_ref, lw_ref):
    j = pl.program_id(0)
    zero = jnp.zeros((H, D // 2), jnp.float32)
    lh_parts = []
    lw_parts = []
    for t in range(OG):
        i = j * OG + t
        posh = rh_ref[pl.ds(31 - i, H), :]
        poswt = jnp.transpose(rw_ref[pl.ds(31 - i, W), :])
        rhs_h = jnp.concatenate([posh, zero], axis=1)
        rhs_w4 = jnp.tile(jnp.concatenate([zero.T, poswt], axis=0), (1, 4))
        xh = qh_ref[:, t].reshape(B * W * G, D)
        xw = qw_ref[:, :, t].reshape(B * H * G, D)
        lh = jax.lax.dot_general(xh, rhs_h, (((1,), (1,)), ((), ())),
                                 preferred_element_type=jnp.float32)
        lw4 = jax.lax.dot_general(xw, rhs_w4, (((1,), (0,)), ((), ())),
                                  preferred_element_type=jnp.float32)
        pad = jnp.zeros((B * W * G, 128 - H), jnp.float32)
        lh_parts.append(jnp.concatenate([lh, pad], axis=1).astype(
            jnp.bfloat16).reshape(B, 1, W * G, 128))
        lw_parts.append(lw4.reshape(B, H, 1, G, 128))
    lh_ref[...] = jnp.concatenate(lh_parts, axis=1)
    lw_ref[...] = jnp.concatenate(lw_parts, axis=2).reshape(
        B, H, OG * G, 128)


def _add_kernel(attn_ref, lh_ref, lw_ref, rep_ref, out_ref):
    rows = HB * W * G
    lh = lh_ref[...].reshape(rows, 128)
    lw4 = lw_ref[...].reshape(rows, 128)
    addend = jax.lax.dot_general(lh, rep_ref[...], (((1,), (0,)), ((), ())),
                                 preferred_element_type=jnp.float32)
    addend += pltpu.repeat(lw4, 8, axis=1)
    out_ref[...] = (attn_ref[...].reshape(rows, H * W) + addend).reshape(
        1, HB * W, G, H * W)


@jax.jit
def kernel(q, attn, rel_emb_h, rel_emb_w):
    QL = H * W
    q5 = q.reshape(B, H, W, G, D)
    rh = jnp.zeros((2 * H, H), jnp.float32).at[: 2 * H - 1].set(rel_emb_h)
    rw = jnp.zeros((2 * W, W), jnp.float32).at[: 2 * W - 1].set(rel_emb_w)

    lh_arr, lw_arr = pl.pallas_call(
        _logits_kernel,
        grid=(H // OG,),
        in_specs=[
            pl.BlockSpec((B, OG, W, G, D), lambda j: (0, j, 0, 0, 0)),
            pl.BlockSpec((B, H, OG, G, D), lambda j: (0, 0, j, 0, 0)),
            pl.BlockSpec((2 * H, H), lambda j: (0, 0)),
            pl.BlockSpec((2 * W, W), lambda j: (0, 0)),
        ],
        out_specs=[
            pl.BlockSpec((B, OG, W * G, 128), lambda j: (0, j, 0, 0)),
            pl.BlockSpec((B, H, OG * G, 128), lambda j: (0, 0, j, 0)),
        ],
        out_shape=[
            jax.ShapeDtypeStruct((B, H, W * G, 128), jnp.bfloat16),
            jax.ShapeDtypeStruct((B, H, W * G, 128), jnp.float32),
        ],
        compiler_params=pltpu.CompilerParams(
            dimension_semantics=("parallel",)),
        name="relpos_logits",
    )(q5, q5, rh, rw)

    j = jnp.arange(QL)
    rep = jnp.zeros((128, QL), jnp.float32).at[:H].set(
        (j[None, :] // W == jnp.arange(H)[:, None]).astype(jnp.float32)
    ).astype(jnp.bfloat16)

    out = pl.pallas_call(
        _add_kernel,
        grid=(B, H // HB),
        in_specs=[
            pl.BlockSpec((1, HB * W, G, QL), lambda b, h: (b, h, 0, 0)),
            pl.BlockSpec((1, HB, W * G, 128), lambda b, h: (b, h, 0, 0)),
            pl.BlockSpec((1, HB, W * G, 128), lambda b, h: (b, h, 0, 0)),
            pl.BlockSpec((128, QL), lambda b, h: (0, 0)),
        ],
        out_specs=pl.BlockSpec((1, HB * W, G, QL), lambda b, h: (b, h, 0, 0)),
        out_shape=jax.ShapeDtypeStruct((B, QL, G, QL), jnp.float32),
        compiler_params=pltpu.CompilerParams(
            dimension_semantics=("parallel", "arbitrary"),
            vmem_limit_bytes=52 * 1024 * 1024),
        name="relpos_add",
    )(attn, lh_arr, lw_arr, rep)
    return out
